# baseline (speedup 1.0000x reference)
import functools

import jax
import jax.numpy as jnp
from jax import lax
from jax.experimental import pallas as pl
from jax.experimental.pallas import tpu as pltpu
from jax.experimental.pallas import tpu_sc as plsc

NCSC = 2
NSUB = 16
NW = NCSC * NSUB
B = 128
D = 128


def _mesh():
    return plsc.VectorSubcoreMesh(
        core_axis_name="c", subcore_axis_name="s",
        num_cores=NCSC, num_subcores=NSUB)


def _make_segsum(n_table, n_dst_pad, nb_main, nb_tail):
    nb = nb_main + nb_tail
    assert nb % 2 == 0
    rpt = n_dst_pad // NSUB
    assert rpt % 128 == 0

    def seg(table, srcm, dstm, srct, dstt, acc_out, deg_out,
            src_v, dst_v, row_a, row_b, ones_v, zdeg,
            acc_sh, deg_sh, sem_a, sem_b, sem_z):
        c = lax.axis_index("c")
        s = lax.axis_index("s")
        w = s * NCSC + c
        base_r = s * rpt

        zero16 = jnp.zeros((16,), jnp.float32)
        one16 = jnp.full((16,), 1.0, jnp.float32)

        def zfill(i, carry):
            for k in range(D // 16):
                row_a[i, pl.ds(16 * k, 16)] = zero16
            return carry
        lax.fori_loop(0, B, zfill, 0)

        def zdfill(i, carry):
            zdeg[pl.ds(16 * i, 16)] = zero16
            return carry
        lax.fori_loop(0, rpt // 16, zdfill, 0)
        for k in range(B // 16):
            ones_v[pl.ds(16 * k, 16)] = one16

        for k in range(rpt // B):
            pltpu.async_copy(row_a, acc_sh.at[pl.ds(base_r + B * k, B)],
                             sem_z)
        pltpu.async_copy(zdeg, deg_sh.at[pl.ds(base_r, rpt)], sem_z)
        pltpu.sync_copy(srcm.at[w], src_v.at[pl.ds(0, nb_main)])
        pltpu.sync_copy(dstm.at[w], dst_v.at[pl.ds(0, nb_main)])
        if nb_tail:
            pltpu.sync_copy(srct.at[w], src_v.at[pl.ds(nb_main, nb_tail)])
            pltpu.sync_copy(dstt.at[w], dst_v.at[pl.ds(nb_main, nb_tail)])
        for k in range(rpt // B):
            pltpu.make_async_copy(row_a, acc_sh.at[pl.ds(base_r, B)],
                                  sem_z).wait()
        pltpu.make_async_copy(zdeg, deg_sh.at[pl.ds(base_r, rpt)],
                              sem_z).wait()

        pltpu.async_copy(table.at[src_v.at[0]], row_b, sem_b)
        plsc.subcore_barrier()

        nh = nb // 2

        def body(i, carry):
            j = 2 * i
            pltpu.async_copy(table.at[src_v.at[j + 1]], row_a, sem_a)
            pltpu.make_async_copy(table.at[src_v.at[0]], row_b, sem_b).wait()
            pltpu.sync_copy(row_b, acc_sh.at[dst_v.at[j]], add=True)
            pltpu.sync_copy(ones_v, deg_sh.at[dst_v.at[j]], add=True)

            @pl.when(i + 1 < nh)
            def _():
                pltpu.async_copy(table.at[src_v.at[j + 2]], row_b, sem_b)

            pltpu.make_async_copy(table.at[src_v.at[0]], row_a, sem_a).wait()
            pltpu.sync_copy(row_a, acc_sh.at[dst_v.at[j + 1]], add=True)
            pltpu.sync_copy(ones_v, deg_sh.at[dst_v.at[j + 1]], add=True)
            return carry

        lax.fori_loop(0, nh, body, 0)
        plsc.subcore_barrier()

        pltpu.sync_copy(acc_sh.at[pl.ds(base_r, rpt)],
                        acc_out.at[c].at[pl.ds(base_r, rpt)])
        pltpu.sync_copy(deg_sh.at[pl.ds(base_r, rpt)],
                        deg_out.at[c].at[pl.ds(base_r, rpt)])

    if not nb_tail:
        def seg_notail(table, srcm, dstm, acc_out, deg_out, *rest):
            return seg(table, srcm, dstm, None, None, acc_out, deg_out, *rest)
        body_fn = seg_notail
    else:
        body_fn = seg

    return functools.partial(
        pl.kernel, mesh=_mesh(),
        out_type=(jax.ShapeDtypeStruct((NCSC, n_dst_pad, D), jnp.float32),
                  jax.ShapeDtypeStruct((NCSC, n_dst_pad), jnp.float32)),
        scratch_types=(
            pltpu.VMEM((nb, B), jnp.int32),
            pltpu.VMEM((nb, B), jnp.int32),
            pltpu.VMEM((B, D), jnp.float32),
            pltpu.VMEM((B, D), jnp.float32),
            pltpu.VMEM((B,), jnp.float32),
            pltpu.VMEM((rpt,), jnp.float32),
            pltpu.VMEM_SHARED((n_dst_pad, D), jnp.float32),
            pltpu.VMEM_SHARED((n_dst_pad,), jnp.float32),
            pltpu.SemaphoreType.DMA,
            pltpu.SemaphoreType.DMA,
            pltpu.SemaphoreType.DMA,
        ),
    )(body_fn)


def _make_gather(n_table, n_idx):
    nb = n_idx // (B * NW)
    assert nb * B * NW == n_idx

    @functools.partial(
        pl.kernel,
        out_type=jax.ShapeDtypeStruct((n_idx, D), jnp.float32),
        mesh=_mesh(),
        scratch_types=(
            pltpu.VMEM((nb, B), jnp.int32),
            pltpu.VMEM((B, D), jnp.float32),
            pltpu.SemaphoreType.DMA,
        ),
    )
    def gat(table, idxm, out, idx_v, row_v, sem):
        c = lax.axis_index("c")
        s = lax.axis_index("s")
        w = s * NCSC + c
        pltpu.sync_copy(idxm.at[w], idx_v)
        for j in range(nb):
            pltpu.async_copy(table.at[idx_v.at[j]], row_v, sem).wait()
            pltpu.sync_copy(row_v, out.at[pl.ds((w * nb + j) * B, B)])

    return gat


def _sage_post(acc, deg, h_prev, w_self, w_neigh, b, g, beta, n_dst):
    n_pad = acc.shape[1]

    def body(acc_ref, deg_ref, h_ref, ws_ref, wn_ref, b_ref, g_ref,
             beta_ref, out_ref):
        agg = acc_ref[0, :n_dst, :] + acc_ref[1, :n_dst, :]
        dg = deg_ref[0, :n_dst] + deg_ref[1, :n_dst]
        mean = agg / jnp.maximum(dg, 1.0)[:, None]
        z = (jnp.dot(h_ref[:n_dst, :], ws_ref[...],
                     preferred_element_type=jnp.float32)
             + jnp.dot(mean, wn_ref[...],
                       preferred_element_type=jnp.float32)
             + b_ref[...])
        mu = jnp.mean(z, axis=0)
        var = jnp.mean((z - mu) ** 2, axis=0)
        zn = (z - mu) * jax.lax.rsqrt(var + 1e-5) * g_ref[...] + beta_ref[...]
        out_ref[...] = jnp.maximum(zn, 0.0)

    return pl.pallas_call(
        body,
        out_shape=jax.ShapeDtypeStruct((n_dst, D), jnp.float32),
    )(acc, deg, h_prev, w_self, w_neigh, b, g, beta)


def _edge_mlp(huv, e_feat, wm1, bm1, wm2, bm2, n_pairs, n_cls):
    def body(huv_ref, ef_ref, w1_ref, b1_ref, w2_ref, b2_ref, out_ref):
        hu = huv_ref[:n_pairs, :]
        hv = huv_ref[n_pairs:, :]
        t = (jnp.dot(hu, w1_ref[:D, :], preferred_element_type=jnp.float32)
             + jnp.dot(hv, w1_ref[D:2 * D, :],
                       preferred_element_type=jnp.float32)
             + jnp.dot(ef_ref[...], w1_ref[2 * D:, :],
                       preferred_element_type=jnp.float32)
             + b1_ref[...])
        t = jnp.maximum(t, 0.0)
        out_ref[...] = (jnp.dot(t, w2_ref[...],
                                preferred_element_type=jnp.float32)
                        + b2_ref[...])

    return pl.pallas_call(
        body,
        out_shape=jax.ShapeDtypeStruct((n_pairs, n_cls), jnp.float32),
    )(huv, e_feat, wm1, bm1, wm2, bm2)


def _split_edges(src, dst, n_table, n_dst, n_dst_pad):
    e = src.shape[0]
    per_w = B * NW
    nb_main = e // per_w
    rem = e - nb_main * per_w
    nb_tail = 0 if rem == 0 else 1
    if (nb_main + nb_tail) % 2:
        if rem == 0:
            nb_main -= 1
            rem = per_w
            nb_tail = 1
        else:
            nb_tail += 1
    e_main = nb_main * per_w
    srcm = src[:e_main].reshape(NW, nb_main, B)
    dstm = dst[:e_main].reshape(NW, nb_main, B)
    if nb_tail == 0:
        return srcm, dstm, None, None, nb_main, 0
    pad = nb_tail * per_w - rem
    ar = jnp.arange(pad, dtype=jnp.int32)
    srct = jnp.concatenate([src[e_main:], ar % n_table])
    dstt = jnp.concatenate([dst[e_main:],
                            n_dst + ar % (n_dst_pad - n_dst)])
    return (srcm, dstm, srct.reshape(NW, nb_tail, B),
            dstt.reshape(NW, nb_tail, B), nb_main, nb_tail)


def kernel(x_nodes, e_feat, W_self0, W_neigh0, b0, g0, beta0,
           W_self1, W_neigh1, b1, g1, beta1, Wm1, bm1, Wm2, bm2,
           edge_index0, edge_index1, pair_edges):
    n0 = x_nodes.shape[0]
    nd0 = 5000
    nd0p = 6144
    nd1 = 2048
    ep = pair_edges.shape[1]

    srcm0, dstm0, srct0, dstt0, nbm0, nbt0 = _split_edges(
        edge_index0[0], edge_index0[1], n0, nd0, nd0p)
    seg0 = _make_segsum(n0, nd0p, nbm0, nbt0)
    args0 = (x_nodes, srcm0, dstm0) + (
        (srct0, dstt0) if nbt0 else ())
    acc0, deg0 = seg0(*args0)
    h0 = _sage_post(acc0, deg0, x_nodes, W_self0, W_neigh0, b0, g0, beta0,
                    nd0)

    srcm1, dstm1, srct1, dstt1, nbm1, nbt1 = _split_edges(
        edge_index1[0], edge_index1[1], nd0, nd1, nd1)
    seg1 = _make_segsum(nd0, nd1, nbm1, nbt1)
    args1 = (h0, srcm1, dstm1) + ((srct1, dstt1) if nbt1 else ())
    acc1, deg1 = seg1(*args1)
    h1 = _sage_post(acc1, deg1, h0, W_self1, W_neigh1, b1, g1, beta1, nd1)

    uvm = jnp.concatenate([pair_edges[0], pair_edges[1]]).reshape(
        NW, 2 * ep // (B * NW), B)
    huv = _make_gather(nd1, 2 * ep)(h1, uvm)
    return _edge_mlp(huv, e_feat, Wm1, bm1, Wm2, bm2, ep, Wm2.shape[1])


# speedup vs baseline: 1.1746x; 1.1746x over previous
import functools

import jax
import jax.numpy as jnp
from jax import lax
from jax.experimental import pallas as pl
from jax.experimental.pallas import tpu as pltpu
from jax.experimental.pallas import tpu_sc as plsc

NCSC = 2
NSUB = 16
NW = NCSC * NSUB
B = 128
D = 128
ICH = 4


def _mesh():
    return plsc.VectorSubcoreMesh(
        core_axis_name="c", subcore_axis_name="s",
        num_cores=NCSC, num_subcores=NSUB)


def _make_segsum(n_table, n_dst_pad, n_batches, nb_max):
    assert nb_max % 2 == 0 and nb_max % ICH == 0
    assert n_batches % 2 == 0 and n_batches % ICH == 0
    rpt = n_dst_pad // NSUB
    assert rpt % 128 == 0

    @functools.partial(
        pl.kernel, mesh=_mesh(),
        out_type=(jax.ShapeDtypeStruct((NCSC, n_dst_pad, D), jnp.float32),
                  jax.ShapeDtypeStruct((NCSC, n_dst_pad), jnp.float32)),
        scratch_types=(
            pltpu.VMEM((2, nb_max * B), jnp.int32),
            pltpu.VMEM((B, D), jnp.float32),
            pltpu.VMEM((B, D), jnp.float32),
            pltpu.VMEM((B,), jnp.float32),
            pltpu.VMEM((rpt,), jnp.float32),
            pltpu.VMEM_SHARED((n_dst_pad, D), jnp.float32),
            pltpu.VMEM_SHARED((n_dst_pad,), jnp.float32),
            pltpu.SemaphoreType.DMA,
            pltpu.SemaphoreType.DMA,
            pltpu.SemaphoreType.DMA,
        ),
    )
    def seg(table, edge, acc_out, deg_out,
            eb, row_a, row_b, ones_v, zdeg, acc_sh, deg_sh,
            sem_a, sem_b, sem_z):
        c = lax.axis_index("c")
        s = lax.axis_index("s")
        w = s * NCSC + c
        base_r = s * rpt
        start_b = w * nb_max
        nbw = jnp.clip(n_batches - start_b, 0, nb_max)

        zero16 = jnp.zeros((16,), jnp.float32)
        one16 = jnp.full((16,), 1.0, jnp.float32)

        def zfill(i, carry):
            for k in range(D // 16):
                row_a[i, pl.ds(16 * k, 16)] = zero16
            return carry
        lax.fori_loop(0, B, zfill, 0)

        def zdfill(i, carry):
            zdeg[pl.ds(16 * i, 16)] = zero16
            return carry
        lax.fori_loop(0, rpt // 16, zdfill, 0)
        for k in range(B // 16):
            ones_v[pl.ds(16 * k, 16)] = one16

        for k in range(rpt // B):
            pltpu.async_copy(row_a, acc_sh.at[pl.ds(base_r + B * k, B)],
                             sem_z)
        pltpu.async_copy(zdeg, deg_sh.at[pl.ds(base_r, rpt)], sem_z)

        def ifetch(k, carry):
            pltpu.async_copy(
                edge.at[:, pl.ds((start_b + k * ICH) * B, ICH * B)],
                eb.at[:, pl.ds(k * ICH * B, ICH * B)], sem_a)
            return carry
        nch = nbw // ICH
        lax.fori_loop(0, nch, ifetch, 0)

        def idrain(k, carry):
            pltpu.make_async_copy(
                edge.at[:, pl.ds(0, ICH * B)],
                eb.at[:, pl.ds(0, ICH * B)], sem_a).wait()
            return carry
        lax.fori_loop(0, nch, idrain, 0)

        for k in range(rpt // B):
            pltpu.make_async_copy(row_a, acc_sh.at[pl.ds(base_r, B)],
                                  sem_z).wait()
        pltpu.make_async_copy(zdeg, deg_sh.at[pl.ds(base_r, rpt)],
                              sem_z).wait()

        @pl.when(nbw > 0)
        def _():
            pltpu.async_copy(table.at[eb.at[0, pl.ds(0, B)]], row_b, sem_b)
        plsc.subcore_barrier()

        def body(i, carry):
            j = 2 * i
            pltpu.async_copy(table.at[eb.at[0, pl.ds((j + 1) * B, B)]],
                             row_a, sem_a)
            pltpu.make_async_copy(table.at[eb.at[0, pl.ds(0, B)]],
                                  row_b, sem_b).wait()
            pltpu.sync_copy(row_b, acc_sh.at[eb.at[1, pl.ds(j * B, B)]],
                            add=True)
            pltpu.sync_copy(ones_v, deg_sh.at[eb.at[1, pl.ds(j * B, B)]],
                            add=True)

            @pl.when(2 * i + 3 < nbw)
            def _():
                pltpu.async_copy(table.at[eb.at[0, pl.ds((j + 2) * B, B)]],
                                 row_b, sem_b)

            pltpu.make_async_copy(table.at[eb.at[0, pl.ds(0, B)]],
                                  row_a, sem_a).wait()
            pltpu.sync_copy(row_a, acc_sh.at[eb.at[1, pl.ds((j + 1) * B, B)]],
                            add=True)
            pltpu.sync_copy(ones_v, deg_sh.at[eb.at[1, pl.ds((j + 1) * B, B)]],
                            add=True)
            return carry

        lax.fori_loop(0, nbw // 2, body, 0)
        plsc.subcore_barrier()

        pltpu.sync_copy(acc_sh.at[pl.ds(base_r, rpt)],
                        acc_out.at[c].at[pl.ds(base_r, rpt)])
        pltpu.sync_copy(deg_sh.at[pl.ds(base_r, rpt)],
                        deg_out.at[c].at[pl.ds(base_r, rpt)])

    return seg


def _make_pair_gather(n_table, ep):
    assert ep == (ep // B) * B and ep // B == NW

    @functools.partial(
        pl.kernel,
        out_type=jax.ShapeDtypeStruct((2 * ep, D), jnp.float32),
        mesh=_mesh(),
        scratch_types=(
            pltpu.VMEM((2, B), jnp.int32),
            pltpu.VMEM((B, D), jnp.float32),
            pltpu.VMEM((B, D), jnp.float32),
            pltpu.SemaphoreType.DMA,
            pltpu.SemaphoreType.DMA,
        ),
    )
    def gat(table, pair, out, eb, row_u, row_v, sem_u, sem_v):
        c = lax.axis_index("c")
        s = lax.axis_index("s")
        w = s * NCSC + c
        pltpu.sync_copy(pair.at[:, pl.ds(w * B, B)], eb)
        pltpu.async_copy(table.at[eb.at[0]], row_u, sem_u)
        pltpu.async_copy(table.at[eb.at[1]], row_v, sem_v)
        pltpu.make_async_copy(table.at[eb.at[0]], row_u, sem_u).wait()
        pltpu.sync_copy(row_u, out.at[pl.ds(w * B, B)])
        pltpu.make_async_copy(table.at[eb.at[1]], row_v, sem_v).wait()
        pltpu.sync_copy(row_v, out.at[pl.ds(ep + w * B, B)])

    return gat


def _sage_post(acc, deg, h_prev, w_self, w_neigh, b, g, beta, n_dst):
    def body(acc_ref, deg_ref, h_ref, ws_ref, wn_ref, b_ref, g_ref,
             beta_ref, out_ref):
        agg = acc_ref[0, :n_dst, :] + acc_ref[1, :n_dst, :]
        dg = deg_ref[0, :n_dst] + deg_ref[1, :n_dst]
        mean = agg / jnp.maximum(dg, 1.0)[:, None]
        z = (jnp.dot(h_ref[:n_dst, :], ws_ref[...],
                     preferred_element_type=jnp.float32)
             + jnp.dot(mean, wn_ref[...],
                       preferred_element_type=jnp.float32)
             + b_ref[...])
        mu = jnp.mean(z, axis=0)
        var = jnp.mean((z - mu) ** 2, axis=0)
        zn = (z - mu) * jax.lax.rsqrt(var + 1e-5) * g_ref[...] + beta_ref[...]
        out_ref[...] = jnp.maximum(zn, 0.0)

    return pl.pallas_call(
        body,
        out_shape=jax.ShapeDtypeStruct((n_dst, D), jnp.float32),
    )(acc, deg, h_prev, w_self, w_neigh, b, g, beta)


def _edge_mlp(huv, e_feat, wm1, bm1, wm2, bm2, n_pairs, n_cls):
    def body(huv_ref, ef_ref, w1_ref, b1_ref, w2_ref, b2_ref, out_ref):
        hu = huv_ref[:n_pairs, :]
        hv = huv_ref[n_pairs:, :]
        t = (jnp.dot(hu, w1_ref[:D, :], preferred_element_type=jnp.float32)
             + jnp.dot(hv, w1_ref[D:2 * D, :],
                       preferred_element_type=jnp.float32)
             + jnp.dot(ef_ref[...], w1_ref[2 * D:, :],
                       preferred_element_type=jnp.float32)
             + b1_ref[...])
        t = jnp.maximum(t, 0.0)
        out_ref[...] = (jnp.dot(t, w2_ref[...],
                                preferred_element_type=jnp.float32)
                        + b2_ref[...])

    return pl.pallas_call(
        body,
        out_shape=jax.ShapeDtypeStruct((n_pairs, n_cls), jnp.float32),
    )(huv, e_feat, wm1, bm1, wm2, bm2)


def kernel(x_nodes, e_feat, W_self0, W_neigh0, b0, g0, beta0,
           W_self1, W_neigh1, b1, g1, beta1, Wm1, bm1, Wm2, bm2,
           edge_index0, edge_index1, pair_edges):
    n0 = x_nodes.shape[0]
    e0 = edge_index0.shape[1]
    e1 = edge_index1.shape[1]
    nd0 = 5000
    nd0p = 6144
    nd1 = 2048
    ep = pair_edges.shape[1]

    acc0, deg0 = _make_segsum(n0, nd0p, e0 // B, 80)(x_nodes, edge_index0)
    h0 = _sage_post(acc0, deg0, x_nodes, W_self0, W_neigh0, b0, g0, beta0,
                    nd0)

    acc1, deg1 = _make_segsum(nd0, nd1, e1 // B, 16)(h0, edge_index1)
    h1 = _sage_post(acc1, deg1, h0, W_self1, W_neigh1, b1, g1, beta1, nd1)

    huv = _make_pair_gather(nd1, ep)(h1, pair_edges)
    return _edge_mlp(huv, e_feat, Wm1, bm1, Wm2, bm2, ep, Wm2.shape[1])


# speedup vs baseline: 1.1777x; 1.0026x over previous
import functools

import jax
import jax.numpy as jnp
from jax import lax
from jax.experimental import pallas as pl
from jax.experimental.pallas import tpu as pltpu
from jax.experimental.pallas import tpu_sc as plsc

NCSC = 2
NSUB = 16
NW = NCSC * NSUB
B = 128
D = 128
ICH = 4


def _mesh():
    return plsc.VectorSubcoreMesh(
        core_axis_name="c", subcore_axis_name="s",
        num_cores=NCSC, num_subcores=NSUB)


def _make_segsum(n_table, n_dst_pad, n_batches, nb_max):
    assert nb_max % 4 == 0 and nb_max % ICH == 0
    assert n_batches % 4 == 0 and n_batches % ICH == 0
    rpt = n_dst_pad // NSUB
    assert rpt % 128 == 0
    hb = nb_max // 2 if nb_max > 16 else nb_max
    assert hb % 4 == 0

    @functools.partial(
        pl.kernel, mesh=_mesh(),
        out_type=(jax.ShapeDtypeStruct((NCSC, n_dst_pad, D), jnp.float32),
                  jax.ShapeDtypeStruct((NCSC, n_dst_pad), jnp.float32)),
        scratch_types=(
            pltpu.VMEM((2, hb * B), jnp.int32),
            pltpu.VMEM((4, B, D), jnp.float32),
            pltpu.VMEM((B,), jnp.float32),
            pltpu.VMEM((rpt,), jnp.float32),
            pltpu.VMEM_SHARED((n_dst_pad, D), jnp.float32),
            pltpu.VMEM_SHARED((n_dst_pad,), jnp.float32),
            pltpu.SemaphoreType.DMA,
            pltpu.SemaphoreType.DMA,
            pltpu.SemaphoreType.DMA,
            pltpu.SemaphoreType.DMA,
        ),
    )
    def seg(table, edge, acc_out, deg_out,
            eb, rows, ones_v, zdeg, acc_sh, deg_sh,
            sem_g, sem_s, sem_d, sem_z):
        c = lax.axis_index("c")
        s = lax.axis_index("s")
        w = s * NCSC + c
        base_r = s * rpt
        start_b = w * nb_max
        nbw = jnp.clip(n_batches - start_b, 0, nb_max)

        zero16 = jnp.zeros((16,), jnp.float32)
        one16 = jnp.full((16,), 1.0, jnp.float32)

        def zfill(i, carry):
            for k in range(D // 16):
                rows[0, i, pl.ds(16 * k, 16)] = zero16
            return carry
        lax.fori_loop(0, B, zfill, 0)

        def zdfill(i, carry):
            zdeg[pl.ds(16 * i, 16)] = zero16
            return carry
        lax.fori_loop(0, rpt // 16, zdfill, 0)
        for k in range(B // 16):
            ones_v[pl.ds(16 * k, 16)] = one16

        for k in range(rpt // B):
            pltpu.async_copy(rows.at[0], acc_sh.at[pl.ds(base_r + B * k, B)],
                             sem_z)
        pltpu.async_copy(zdeg, deg_sh.at[pl.ds(base_r, rpt)], sem_z)

        cnt_a = jnp.minimum(nbw, hb)
        cnt_b = nbw - cnt_a

        def fetch_idx(first_b, cnt):
            def f(k, carry):
                pltpu.async_copy(
                    edge.at[:, pl.ds((first_b + k * ICH) * B, ICH * B)],
                    eb.at[:, pl.ds(k * ICH * B, ICH * B)], sem_d)
                return carry
            lax.fori_loop(0, cnt // ICH, f, 0)

            def fd(k, carry):
                pltpu.make_async_copy(
                    edge.at[:, pl.ds(0, ICH * B)],
                    eb.at[:, pl.ds(0, ICH * B)], sem_d).wait()
                return carry
            lax.fori_loop(0, cnt // ICH, fd, 0)

        def run_batches(cnt):
            for k in range(4):
                @pl.when(k < cnt)
                def _():
                    pltpu.async_copy(table.at[eb.at[0, pl.ds(k * B, B)]],
                                     rows.at[k], sem_g)

            def body(i, carry):
                b0 = 4 * i
                for k in range(4):
                    pltpu.make_async_copy(table.at[eb.at[0, pl.ds(0, B)]],
                                          rows.at[k], sem_g).wait()
                    pltpu.async_copy(
                        rows.at[k],
                        acc_sh.at[eb.at[1, pl.ds((b0 + k) * B, B)]],
                        sem_s, add=True)
                    pltpu.async_copy(
                        ones_v,
                        deg_sh.at[eb.at[1, pl.ds((b0 + k) * B, B)]],
                        sem_d, add=True)
                for k in range(4):
                    pltpu.make_async_copy(rows.at[k], acc_sh.at[pl.ds(0, B)],
                                          sem_s).wait()
                    pltpu.make_async_copy(ones_v, deg_sh.at[pl.ds(0, B)],
                                          sem_d).wait()

                    @pl.when(b0 + 4 + k < cnt)
                    def _():
                        pltpu.async_copy(
                            table.at[eb.at[0, pl.ds((b0 + 4 + k) * B, B)]],
                            rows.at[k], sem_g)
                return carry

            lax.fori_loop(0, cnt // 4, body, 0)

        fetch_idx(start_b, cnt_a)
        for k in range(rpt // B):
            pltpu.make_async_copy(rows.at[0], acc_sh.at[pl.ds(base_r, B)],
                                  sem_z).wait()
        pltpu.make_async_copy(zdeg, deg_sh.at[pl.ds(base_r, rpt)],
                              sem_z).wait()
        plsc.subcore_barrier()
        run_batches(cnt_a)
        if hb < nb_max:
            fetch_idx(start_b + hb, cnt_b)
            run_batches(cnt_b)
        plsc.subcore_barrier()

        pltpu.sync_copy(acc_sh.at[pl.ds(base_r, rpt)],
                        acc_out.at[c].at[pl.ds(base_r, rpt)])
        pltpu.sync_copy(deg_sh.at[pl.ds(base_r, rpt)],
                        deg_out.at[c].at[pl.ds(base_r, rpt)])

    return seg


def _make_pair_gather(n_table, ep):
    assert ep == (ep // B) * B and ep // B == NW

    @functools.partial(
        pl.kernel,
        out_type=jax.ShapeDtypeStruct((2 * ep, D), jnp.float32),
        mesh=_mesh(),
        scratch_types=(
            pltpu.VMEM((2, B), jnp.int32),
            pltpu.VMEM((B, D), jnp.float32),
            pltpu.VMEM((B, D), jnp.float32),
            pltpu.SemaphoreType.DMA,
            pltpu.SemaphoreType.DMA,
        ),
    )
    def gat(table, pair, out, eb, row_u, row_v, sem_u, sem_v):
        c = lax.axis_index("c")
        s = lax.axis_index("s")
        w = s * NCSC + c
        pltpu.sync_copy(pair.at[:, pl.ds(w * B, B)], eb)
        pltpu.async_copy(table.at[eb.at[0]], row_u, sem_u)
        pltpu.async_copy(table.at[eb.at[1]], row_v, sem_v)
        pltpu.make_async_copy(table.at[eb.at[0]], row_u, sem_u).wait()
        pltpu.sync_copy(row_u, out.at[pl.ds(w * B, B)])
        pltpu.make_async_copy(table.at[eb.at[1]], row_v, sem_v).wait()
        pltpu.sync_copy(row_v, out.at[pl.ds(ep + w * B, B)])

    return gat


def _sage_post(acc, deg, h_prev, w_self, w_neigh, b, g, beta, n_dst):
    def body(acc_ref, deg_ref, h_ref, ws_ref, wn_ref, b_ref, g_ref,
             beta_ref, out_ref):
        agg = acc_ref[0, :n_dst, :] + acc_ref[1, :n_dst, :]
        dg = deg_ref[0, :n_dst] + deg_ref[1, :n_dst]
        mean = agg / jnp.maximum(dg, 1.0)[:, None]
        z = (jnp.dot(h_ref[:n_dst, :], ws_ref[...],
                     preferred_element_type=jnp.float32)
             + jnp.dot(mean, wn_ref[...],
                       preferred_element_type=jnp.float32)
             + b_ref[...])
        mu = jnp.mean(z, axis=0)
        var = jnp.mean((z - mu) ** 2, axis=0)
        zn = (z - mu) * jax.lax.rsqrt(var + 1e-5) * g_ref[...] + beta_ref[...]
        out_ref[...] = jnp.maximum(zn, 0.0)

    return pl.pallas_call(
        body,
        out_shape=jax.ShapeDtypeStruct((n_dst, D), jnp.float32),
    )(acc, deg, h_prev, w_self, w_neigh, b, g, beta)


def _edge_mlp(huv, e_feat, wm1, bm1, wm2, bm2, n_pairs, n_cls):
    def body(huv_ref, ef_ref, w1_ref, b1_ref, w2_ref, b2_ref, out_ref):
        hu = huv_ref[:n_pairs, :]
        hv = huv_ref[n_pairs:, :]
        t = (jnp.dot(hu, w1_ref[:D, :], preferred_element_type=jnp.float32)
             + jnp.dot(hv, w1_ref[D:2 * D, :],
                       preferred_element_type=jnp.float32)
             + jnp.dot(ef_ref[...], w1_ref[2 * D:, :],
                       preferred_element_type=jnp.float32)
             + b1_ref[...])
        t = jnp.maximum(t, 0.0)
        out_ref[...] = (jnp.dot(t, w2_ref[...],
                                preferred_element_type=jnp.float32)
                        + b2_ref[...])

    return pl.pallas_call(
        body,
        out_shape=jax.ShapeDtypeStruct((n_pairs, n_cls), jnp.float32),
    )(huv, e_feat, wm1, bm1, wm2, bm2)


def kernel(x_nodes, e_feat, W_self0, W_neigh0, b0, g0, beta0,
           W_self1, W_neigh1, b1, g1, beta1, Wm1, bm1, Wm2, bm2,
           edge_index0, edge_index1, pair_edges):
    n0 = x_nodes.shape[0]
    e0 = edge_index0.shape[1]
    e1 = edge_index1.shape[1]
    nd0 = 5000
    nd0p = 6144
    nd1 = 2048
    ep = pair_edges.shape[1]

    acc0, deg0 = _make_segsum(n0, nd0p, e0 // B, 80)(x_nodes, edge_index0)
    h0 = _sage_post(acc0, deg0, x_nodes, W_self0, W_neigh0, b0, g0, beta0,
                    nd0)

    acc1, deg1 = _make_segsum(nd0, nd1, e1 // B, 16)(h0, edge_index1)
    h1 = _sage_post(acc1, deg1, h0, W_self1, W_neigh1, b1, g1, beta1, nd1)

    huv = _make_pair_gather(nd1, ep)(h1, pair_edges)
    return _edge_mlp(huv, e_feat, Wm1, bm1, Wm2, bm2, ep, Wm2.shape[1])


# speedup vs baseline: 1.2145x; 1.0313x over previous
import functools

import jax
import jax.numpy as jnp
from jax import lax
from jax.experimental import pallas as pl
from jax.experimental.pallas import tpu as pltpu
from jax.experimental.pallas import tpu_sc as plsc

NCSC = 2
NSUB = 16
NW = NCSC * NSUB
B = 128
D = 128
ICH = 4


def _mesh():
    return plsc.VectorSubcoreMesh(
        core_axis_name="c", subcore_axis_name="s",
        num_cores=NCSC, num_subcores=NSUB)


def _make_segsum(n_table, n_dst_pad, n_batches, nb_max):
    assert nb_max % 4 == 0 and nb_max % ICH == 0
    assert n_batches % 4 == 0 and n_batches % ICH == 0
    rpt = n_dst_pad // NSUB
    assert rpt % 128 == 0
    hb = nb_max // 2 if nb_max > 16 else nb_max
    assert hb % 4 == 0

    @functools.partial(
        pl.kernel, mesh=_mesh(),
        out_type=(jax.ShapeDtypeStruct((NCSC, n_dst_pad, D), jnp.float32),
                  jax.ShapeDtypeStruct((NCSC, n_dst_pad), jnp.float32)),
        scratch_types=(
            pltpu.VMEM((2, hb * B), jnp.int32),
            pltpu.VMEM((4, B, D), jnp.float32),
            pltpu.VMEM((B,), jnp.float32),
            pltpu.VMEM((rpt,), jnp.float32),
            pltpu.VMEM_SHARED((n_dst_pad, D), jnp.float32),
            pltpu.VMEM_SHARED((n_dst_pad,), jnp.float32),
            pltpu.SemaphoreType.DMA,
            pltpu.SemaphoreType.DMA,
            pltpu.SemaphoreType.DMA,
            pltpu.SemaphoreType.DMA,
        ),
    )
    def seg(table, edge, acc_out, deg_out,
            eb, rows, ones_v, zdeg, acc_sh, deg_sh,
            sem_g, sem_s, sem_d, sem_z):
        c = lax.axis_index("c")
        s = lax.axis_index("s")
        w = s * NCSC + c
        base_r = s * rpt
        start_b = w * nb_max
        nbw = jnp.clip(n_batches - start_b, 0, nb_max)

        zero16 = jnp.zeros((16,), jnp.float32)
        one16 = jnp.full((16,), 1.0, jnp.float32)

        def zfill(i, carry):
            for k in range(D // 16):
                rows[0, i, pl.ds(16 * k, 16)] = zero16
            return carry
        lax.fori_loop(0, B, zfill, 0)

        def zdfill(i, carry):
            zdeg[pl.ds(16 * i, 16)] = zero16
            return carry
        lax.fori_loop(0, rpt // 16, zdfill, 0)
        for k in range(B // 16):
            ones_v[pl.ds(16 * k, 16)] = one16

        for k in range(rpt // B):
            pltpu.async_copy(rows.at[0], acc_sh.at[pl.ds(base_r + B * k, B)],
                             sem_z)
        pltpu.async_copy(zdeg, deg_sh.at[pl.ds(base_r, rpt)], sem_z)

        cnt_a = jnp.minimum(nbw, hb)
        cnt_b = nbw - cnt_a

        def fetch_idx(first_b, cnt):
            def f(k, carry):
                pltpu.async_copy(
                    edge.at[:, pl.ds((first_b + k * ICH) * B, ICH * B)],
                    eb.at[:, pl.ds(k * ICH * B, ICH * B)], sem_d)
                return carry
            lax.fori_loop(0, cnt // ICH, f, 0)

            def fd(k, carry):
                pltpu.make_async_copy(
                    edge.at[:, pl.ds(0, ICH * B)],
                    eb.at[:, pl.ds(0, ICH * B)], sem_d).wait()
                return carry
            lax.fori_loop(0, cnt // ICH, fd, 0)

        def run_batches(cnt):
            for k in range(4):
                @pl.when(k < cnt)
                def _():
                    pltpu.async_copy(table.at[eb.at[0, pl.ds(k * B, B)]],
                                     rows.at[k], sem_g)

            def body(i, carry):
                b0 = 4 * i
                for k in range(4):
                    pltpu.make_async_copy(table.at[eb.at[0, pl.ds(0, B)]],
                                          rows.at[k], sem_g).wait()
                    pltpu.async_copy(
                        rows.at[k],
                        acc_sh.at[eb.at[1, pl.ds((b0 + k) * B, B)]],
                        sem_s, add=True)
                    pltpu.async_copy(
                        ones_v,
                        deg_sh.at[eb.at[1, pl.ds((b0 + k) * B, B)]],
                        sem_d, add=True)
                for k in range(4):
                    pltpu.make_async_copy(rows.at[k], acc_sh.at[pl.ds(0, B)],
                                          sem_s).wait()
                    pltpu.make_async_copy(ones_v, deg_sh.at[pl.ds(0, B)],
                                          sem_d).wait()

                    @pl.when(b0 + 4 + k < cnt)
                    def _():
                        pltpu.async_copy(
                            table.at[eb.at[0, pl.ds((b0 + 4 + k) * B, B)]],
                            rows.at[k], sem_g)
                return carry

            lax.fori_loop(0, cnt // 4, body, 0)

        fetch_idx(start_b, cnt_a)
        for k in range(rpt // B):
            pltpu.make_async_copy(rows.at[0], acc_sh.at[pl.ds(base_r, B)],
                                  sem_z).wait()
        pltpu.make_async_copy(zdeg, deg_sh.at[pl.ds(base_r, rpt)],
                              sem_z).wait()
        plsc.subcore_barrier()
        run_batches(cnt_a)
        if hb < nb_max:
            fetch_idx(start_b + hb, cnt_b)
            run_batches(cnt_b)
        plsc.subcore_barrier()

        pltpu.sync_copy(acc_sh.at[pl.ds(base_r, rpt)],
                        acc_out.at[c].at[pl.ds(base_r, rpt)])
        pltpu.sync_copy(deg_sh.at[pl.ds(base_r, rpt)],
                        deg_out.at[c].at[pl.ds(base_r, rpt)])

    return seg


def _make_pair_gather(n_table, ep):
    assert ep == (ep // B) * B and ep // B == NW

    @functools.partial(
        pl.kernel,
        out_type=jax.ShapeDtypeStruct((2 * ep, D), jnp.float32),
        mesh=_mesh(),
        scratch_types=(
            pltpu.VMEM((2, B), jnp.int32),
            pltpu.VMEM((B, D), jnp.float32),
            pltpu.VMEM((B, D), jnp.float32),
            pltpu.SemaphoreType.DMA,
            pltpu.SemaphoreType.DMA,
        ),
    )
    def gat(table, pair, out, eb, row_u, row_v, sem_u, sem_v):
        c = lax.axis_index("c")
        s = lax.axis_index("s")
        w = s * NCSC + c
        pltpu.sync_copy(pair.at[:, pl.ds(w * B, B)], eb)
        pltpu.async_copy(table.at[eb.at[0]], row_u, sem_u)
        pltpu.async_copy(table.at[eb.at[1]], row_v, sem_v)
        pltpu.make_async_copy(table.at[eb.at[0]], row_u, sem_u).wait()
        pltpu.sync_copy(row_u, out.at[pl.ds(w * B, B)])
        pltpu.make_async_copy(table.at[eb.at[1]], row_v, sem_v).wait()
        pltpu.sync_copy(row_v, out.at[pl.ds(ep + w * B, B)])

    return gat


def _self_proj(h_prev, w_self, b, n_dst):
    def body(h_ref, ws_ref, b_ref, out_ref):
        out_ref[...] = (jnp.dot(h_ref[:n_dst, :], ws_ref[...],
                                preferred_element_type=jnp.float32)
                        + b_ref[...])

    return pl.pallas_call(
        body,
        out_shape=jax.ShapeDtypeStruct((n_dst, D), jnp.float32),
    )(h_prev, w_self, b)


def _sage_post(acc, deg, zs, w_neigh, g, beta, n_dst):
    def body(acc_ref, deg_ref, zs_ref, wn_ref, g_ref, beta_ref, out_ref):
        agg = acc_ref[0, :n_dst, :] + acc_ref[1, :n_dst, :]
        dg = deg_ref[0, :n_dst] + deg_ref[1, :n_dst]
        mean = agg / jnp.maximum(dg, 1.0)[:, None]
        z = zs_ref[...] + jnp.dot(mean, wn_ref[...],
                                  preferred_element_type=jnp.float32)
        mu = jnp.mean(z, axis=0)
        var = jnp.mean((z - mu) ** 2, axis=0)
        zn = (z - mu) * jax.lax.rsqrt(var + 1e-5) * g_ref[...] + beta_ref[...]
        out_ref[...] = jnp.maximum(zn, 0.0)

    return pl.pallas_call(
        body,
        out_shape=jax.ShapeDtypeStruct((n_dst, D), jnp.float32),
    )(acc, deg, zs, w_neigh, g, beta)


def _edge_mlp(huv, e_feat_t, wm1, bm1, wm2, bm2, n_pairs, n_cls):
    def body(huv_ref, eft_ref, w1_ref, b1_ref, w2_ref, b2_ref, out_ref):
        hu = huv_ref[:n_pairs, :]
        hv = huv_ref[n_pairs:, :]
        t = (jnp.dot(hu, w1_ref[:D, :], preferred_element_type=jnp.float32)
             + jnp.dot(hv, w1_ref[D:2 * D, :],
                       preferred_element_type=jnp.float32)
             + lax.dot_general(eft_ref[...], w1_ref[2 * D:, :],
                               (((0,), (0,)), ((), ())),
                               preferred_element_type=jnp.float32)
             + b1_ref[...])
        t = jnp.maximum(t, 0.0)
        out_ref[...] = (lax.dot_general(w2_ref[...], t,
                                        (((0,), (1,)), ((), ())),
                                        preferred_element_type=jnp.float32)
                        + b2_ref[...][:, None])

    return pl.pallas_call(
        body,
        out_shape=jax.ShapeDtypeStruct((n_cls, n_pairs), jnp.float32),
    )(huv, e_feat_t, wm1, bm1, wm2, bm2)


def kernel(x_nodes, e_feat, W_self0, W_neigh0, b0, g0, beta0,
           W_self1, W_neigh1, b1, g1, beta1, Wm1, bm1, Wm2, bm2,
           edge_index0, edge_index1, pair_edges):
    n0 = x_nodes.shape[0]
    e0 = edge_index0.shape[1]
    e1 = edge_index1.shape[1]
    nd0 = 5000
    nd0p = 6144
    nd1 = 2048
    ep = pair_edges.shape[1]

    zs0 = _self_proj(x_nodes, W_self0, b0, nd0)
    acc0, deg0 = _make_segsum(n0, nd0p, e0 // B, 80)(x_nodes, edge_index0)
    h0 = _sage_post(acc0, deg0, zs0, W_neigh0, g0, beta0, nd0)

    zs1 = _self_proj(h0, W_self1, b1, nd1)
    acc1, deg1 = _make_segsum(nd0, nd1, e1 // B, 16)(h0, edge_index1)
    h1 = _sage_post(acc1, deg1, zs1, W_neigh1, g1, beta1, nd1)

    huv = _make_pair_gather(nd1, ep)(h1, pair_edges)
    out_t = _edge_mlp(huv, e_feat.T, Wm1, bm1, Wm2, bm2, ep, Wm2.shape[1])
    return out_t.T


# speedup vs baseline: 1.2506x; 1.0297x over previous
import functools

import jax
import jax.numpy as jnp
from jax import lax
from jax.experimental import pallas as pl
from jax.experimental.pallas import tpu as pltpu
from jax.experimental.pallas import tpu_sc as plsc

NCSC = 2
NSUB = 16
NW = NCSC * NSUB
B = 128
D = 128
ICH = 4


def _mesh():
    return plsc.VectorSubcoreMesh(
        core_axis_name="c", subcore_axis_name="s",
        num_cores=NCSC, num_subcores=NSUB)


def _make_segsum(n_table, n_dst_pad, n_batches, nb_max):
    assert nb_max % 4 == 0 and nb_max % ICH == 0
    assert n_batches % 4 == 0 and n_batches % ICH == 0
    rpt = n_dst_pad // NSUB
    assert rpt % 8 == 0
    if rpt % 128 == 0:
        dchunk, dtiles = rpt, NSUB
    else:
        assert n_dst_pad % (8 * 128) == 0
        dchunk, dtiles = n_dst_pad // 8, 8
    hb = nb_max
    assert hb % 4 == 0

    @functools.partial(
        pl.kernel, mesh=_mesh(),
        out_type=(jax.ShapeDtypeStruct((NCSC, n_dst_pad, D), jnp.float32),
                  jax.ShapeDtypeStruct((NCSC, n_dst_pad), jnp.float32)),
        scratch_types=(
            pltpu.VMEM((2, hb * B), jnp.int32),
            pltpu.VMEM((4, B, D), jnp.float32),
            pltpu.VMEM((B,), jnp.float32),
            pltpu.VMEM((dchunk,), jnp.float32),
            pltpu.VMEM_SHARED((n_dst_pad, D), jnp.float32),
            pltpu.VMEM_SHARED((n_dst_pad,), jnp.float32),
            pltpu.SemaphoreType.DMA,
            pltpu.SemaphoreType.DMA,
            pltpu.SemaphoreType.DMA,
            pltpu.SemaphoreType.DMA,
        ),
    )
    def seg(table, edge, acc_out, deg_out,
            eb, rows, ones_v, zdeg, acc_sh, deg_sh,
            sem_g, sem_s, sem_d, sem_z):
        c = lax.axis_index("c")
        s = lax.axis_index("s")
        w = s * NCSC + c
        base_r = s * rpt
        start_b = w * nb_max
        nbw = jnp.clip(n_batches - start_b, 0, nb_max)

        zero16 = jnp.zeros((16,), jnp.float32)
        one16 = jnp.full((16,), 1.0, jnp.float32)

        def zfill(i, carry):
            for k in range(D // 16):
                rows[0, i, pl.ds(16 * k, 16)] = zero16
            return carry
        lax.fori_loop(0, B, zfill, 0)

        def zdfill(i, carry):
            zdeg[pl.ds(16 * i, 16)] = zero16
            return carry
        lax.fori_loop(0, dchunk // 16, zdfill, 0)
        for k in range(B // 16):
            ones_v[pl.ds(16 * k, 16)] = one16

        for k in range(rpt // B):
            pltpu.async_copy(rows.at[0], acc_sh.at[pl.ds(base_r + B * k, B)],
                             sem_z)
        if rpt % B:
            pltpu.async_copy(
                rows.at[0].at[pl.ds(0, rpt % B)],
                acc_sh.at[pl.ds(base_r + (rpt // B) * B, rpt % B)], sem_z)

        @pl.when(s < dtiles)
        def _():
            pltpu.async_copy(zdeg, deg_sh.at[pl.ds(s * dchunk, dchunk)],
                             sem_z)

        cnt_a = jnp.minimum(nbw, hb)
        cnt_b = nbw - cnt_a

        def fetch_idx(first_b, cnt):
            def f(k, carry):
                pltpu.async_copy(
                    edge.at[:, pl.ds((first_b + k * ICH) * B, ICH * B)],
                    eb.at[:, pl.ds(k * ICH * B, ICH * B)], sem_d)
                return carry
            lax.fori_loop(0, cnt // ICH, f, 0)

            def fd(k, carry):
                pltpu.make_async_copy(
                    edge.at[:, pl.ds(0, ICH * B)],
                    eb.at[:, pl.ds(0, ICH * B)], sem_d).wait()
                return carry
            lax.fori_loop(0, cnt // ICH, fd, 0)

        def run_batches(cnt):
            for k in range(4):
                @pl.when(k < cnt)
                def _():
                    pltpu.async_copy(table.at[eb.at[0, pl.ds(k * B, B)]],
                                     rows.at[k], sem_g)

            def body(i, carry):
                b0 = 4 * i
                for k in range(4):
                    pltpu.make_async_copy(table.at[eb.at[0, pl.ds(0, B)]],
                                          rows.at[k], sem_g).wait()
                    pltpu.async_copy(
                        rows.at[k],
                        acc_sh.at[eb.at[1, pl.ds((b0 + k) * B, B)]],
                        sem_s, add=True)
                    pltpu.async_copy(
                        ones_v,
                        deg_sh.at[eb.at[1, pl.ds((b0 + k) * B, B)]],
                        sem_d, add=True)
                for k in range(4):
                    pltpu.make_async_copy(rows.at[k], acc_sh.at[pl.ds(0, B)],
                                          sem_s).wait()
                    pltpu.make_async_copy(ones_v, deg_sh.at[pl.ds(0, B)],
                                          sem_d).wait()

                    @pl.when(b0 + 4 + k < cnt)
                    def _():
                        pltpu.async_copy(
                            table.at[eb.at[0, pl.ds((b0 + 4 + k) * B, B)]],
                            rows.at[k], sem_g)
                return carry

            lax.fori_loop(0, cnt // 4, body, 0)

        fetch_idx(start_b, cnt_a)
        for k in range(rpt // B):
            pltpu.make_async_copy(rows.at[0], acc_sh.at[pl.ds(base_r, B)],
                                  sem_z).wait()
        if rpt % B:
            pltpu.make_async_copy(
                rows.at[0].at[pl.ds(0, rpt % B)],
                acc_sh.at[pl.ds(base_r, rpt % B)], sem_z).wait()

        @pl.when(s < dtiles)
        def _():
            pltpu.make_async_copy(zdeg, deg_sh.at[pl.ds(0, dchunk)],
                                  sem_z).wait()
        plsc.subcore_barrier()
        run_batches(cnt_a)
        if hb < nb_max:
            fetch_idx(start_b + hb, cnt_b)
            run_batches(cnt_b)
        plsc.subcore_barrier()

        pltpu.sync_copy(acc_sh.at[pl.ds(base_r, rpt)],
                        acc_out.at[c].at[pl.ds(base_r, rpt)])

        @pl.when(s < dtiles)
        def _():
            pltpu.sync_copy(deg_sh.at[pl.ds(s * dchunk, dchunk)],
                            deg_out.at[c].at[pl.ds(s * dchunk, dchunk)])

    return seg


def _make_pair_gather(n_table, ep):
    assert ep == (ep // B) * B and ep // B == NW

    @functools.partial(
        pl.kernel,
        out_type=jax.ShapeDtypeStruct((2 * ep, D), jnp.float32),
        mesh=_mesh(),
        scratch_types=(
            pltpu.VMEM((2, B), jnp.int32),
            pltpu.VMEM((B, D), jnp.float32),
            pltpu.VMEM((B, D), jnp.float32),
            pltpu.SemaphoreType.DMA,
            pltpu.SemaphoreType.DMA,
        ),
    )
    def gat(table, pair, out, eb, row_u, row_v, sem_u, sem_v):
        c = lax.axis_index("c")
        s = lax.axis_index("s")
        w = s * NCSC + c
        pltpu.sync_copy(pair.at[:, pl.ds(w * B, B)], eb)
        pltpu.async_copy(table.at[eb.at[0]], row_u, sem_u)
        pltpu.async_copy(table.at[eb.at[1]], row_v, sem_v)
        pltpu.make_async_copy(table.at[eb.at[0]], row_u, sem_u).wait()
        pltpu.sync_copy(row_u, out.at[pl.ds(w * B, B)])
        pltpu.make_async_copy(table.at[eb.at[1]], row_v, sem_v).wait()
        pltpu.sync_copy(row_v, out.at[pl.ds(ep + w * B, B)])

    return gat


def _self_proj(h_prev, w_self, b, n_dst):
    def body(h_ref, ws_ref, b_ref, out_ref):
        out_ref[...] = (jnp.dot(h_ref[:n_dst, :], ws_ref[...],
                                preferred_element_type=jnp.float32)
                        + b_ref[...])

    return pl.pallas_call(
        body,
        out_shape=jax.ShapeDtypeStruct((n_dst, D), jnp.float32),
    )(h_prev, w_self, b)


def _sage_post(acc, deg, zs, w_neigh, g, beta, n_dst):
    def body(acc_ref, deg_ref, zs_ref, wn_ref, g_ref, beta_ref, out_ref):
        agg = acc_ref[0, :n_dst, :] + acc_ref[1, :n_dst, :]
        dg = deg_ref[0, :n_dst] + deg_ref[1, :n_dst]
        mean = agg / jnp.maximum(dg, 1.0)[:, None]
        z = zs_ref[...] + jnp.dot(mean, wn_ref[...],
                                  preferred_element_type=jnp.float32)
        mu = jnp.mean(z, axis=0)
        var = jnp.mean((z - mu) ** 2, axis=0)
        zn = (z - mu) * jax.lax.rsqrt(var + 1e-5) * g_ref[...] + beta_ref[...]
        out_ref[...] = jnp.maximum(zn, 0.0)

    return pl.pallas_call(
        body,
        out_shape=jax.ShapeDtypeStruct((n_dst, D), jnp.float32),
    )(acc, deg, zs, w_neigh, g, beta)


def _edge_mlp(huv, e_feat_t, wm1, bm1, wm2, bm2, n_pairs, n_cls):
    def body(huv_ref, eft_ref, w1_ref, b1_ref, w2_ref, b2_ref, out_ref):
        hu = huv_ref[:n_pairs, :]
        hv = huv_ref[n_pairs:, :]
        t = (jnp.dot(hu, w1_ref[:D, :], preferred_element_type=jnp.float32)
             + jnp.dot(hv, w1_ref[D:2 * D, :],
                       preferred_element_type=jnp.float32)
             + lax.dot_general(eft_ref[...], w1_ref[2 * D:, :],
                               (((0,), (0,)), ((), ())),
                               preferred_element_type=jnp.float32)
             + b1_ref[...])
        t = jnp.maximum(t, 0.0)
        out_ref[...] = (lax.dot_general(w2_ref[...], t,
                                        (((0,), (1,)), ((), ())),
                                        preferred_element_type=jnp.float32)
                        + b2_ref[...][:, None])

    return pl.pallas_call(
        body,
        out_shape=jax.ShapeDtypeStruct((n_cls, n_pairs), jnp.float32),
    )(huv, e_feat_t, wm1, bm1, wm2, bm2)


def kernel(x_nodes, e_feat, W_self0, W_neigh0, b0, g0, beta0,
           W_self1, W_neigh1, b1, g1, beta1, Wm1, bm1, Wm2, bm2,
           edge_index0, edge_index1, pair_edges):
    n0 = x_nodes.shape[0]
    e0 = edge_index0.shape[1]
    e1 = edge_index1.shape[1]
    nd0 = 5000
    nd0p = 5120
    nd1 = 2048
    ep = pair_edges.shape[1]

    zs0 = _self_proj(x_nodes, W_self0, b0, nd0)
    acc0, deg0 = _make_segsum(n0, nd0p, e0 // B, 80)(x_nodes, edge_index0)
    h0 = _sage_post(acc0, deg0, zs0, W_neigh0, g0, beta0, nd0)

    zs1 = _self_proj(h0, W_self1, b1, nd1)
    acc1, deg1 = _make_segsum(nd0, nd1, e1 // B, 16)(h0, edge_index1)
    h1 = _sage_post(acc1, deg1, zs1, W_neigh1, g1, beta1, nd1)

    huv = _make_pair_gather(nd1, ep)(h1, pair_edges)
    out_t = _edge_mlp(huv, e_feat.T, Wm1, bm1, Wm2, bm2, ep, Wm2.shape[1])
    return out_t.T


# speedup vs baseline: 1.2524x; 1.0014x over previous
import functools

import jax
import jax.numpy as jnp
from jax import lax
from jax.experimental import pallas as pl
from jax.experimental.pallas import tpu as pltpu
from jax.experimental.pallas import tpu_sc as plsc

NCSC = 2
NSUB = 16
NW = NCSC * NSUB
B = 128
D = 128
ICH = 4


def _mesh():
    return plsc.VectorSubcoreMesh(
        core_axis_name="c", subcore_axis_name="s",
        num_cores=NCSC, num_subcores=NSUB)


def _make_segsum(n_table, n_dst_pad, n_batches, nb_max):
    assert nb_max % 4 == 0 and nb_max % ICH == 0
    assert n_batches % 4 == 0 and n_batches % ICH == 0
    rpt = n_dst_pad // NSUB
    assert rpt % 8 == 0
    if rpt % 128 == 0:
        dchunk, dtiles = rpt, NSUB
    else:
        assert n_dst_pad % (8 * 128) == 0
        dchunk, dtiles = n_dst_pad // 8, 8
    hb = nb_max
    assert hb % 4 == 0

    @functools.partial(
        pl.kernel, mesh=_mesh(),
        out_type=(jax.ShapeDtypeStruct((NCSC, n_dst_pad, D), jnp.float32),
                  jax.ShapeDtypeStruct((NCSC, n_dst_pad), jnp.float32)),
        scratch_types=(
            pltpu.VMEM((2, hb * B), jnp.int32),
            pltpu.VMEM((4, B, D), jnp.float32),
            pltpu.VMEM((B,), jnp.float32),
            pltpu.VMEM((dchunk,), jnp.float32),
            pltpu.VMEM_SHARED((n_dst_pad, D), jnp.float32),
            pltpu.VMEM_SHARED((n_dst_pad,), jnp.float32),
            pltpu.SemaphoreType.DMA,
            pltpu.SemaphoreType.DMA,
            pltpu.SemaphoreType.DMA,
            pltpu.SemaphoreType.DMA,
        ),
    )
    def seg(table, edge, acc_out, deg_out,
            eb, rows, ones_v, zdeg, acc_sh, deg_sh,
            sem_g, sem_s, sem_d, sem_z):
        c = lax.axis_index("c")
        s = lax.axis_index("s")
        w = s * NCSC + c
        base_r = s * rpt
        start_b = w * nb_max
        nbw = jnp.clip(n_batches - start_b, 0, nb_max)

        zero16 = jnp.zeros((16,), jnp.float32)
        one16 = jnp.full((16,), 1.0, jnp.float32)

        def zfill(i, carry):
            for k in range(D // 16):
                rows[3, i, pl.ds(16 * k, 16)] = zero16
            return carry
        lax.fori_loop(0, B, zfill, 0)

        def zdfill(i, carry):
            zdeg[pl.ds(16 * i, 16)] = zero16
            return carry
        lax.fori_loop(0, dchunk // 16, zdfill, 0)
        for k in range(B // 16):
            ones_v[pl.ds(16 * k, 16)] = one16

        for k in range(rpt // B):
            pltpu.async_copy(rows.at[3], acc_sh.at[pl.ds(base_r + B * k, B)],
                             sem_z)
        if rpt % B:
            pltpu.async_copy(
                rows.at[3].at[pl.ds(0, rpt % B)],
                acc_sh.at[pl.ds(base_r + (rpt // B) * B, rpt % B)], sem_z)

        @pl.when(s < dtiles)
        def _():
            pltpu.async_copy(zdeg, deg_sh.at[pl.ds(s * dchunk, dchunk)],
                             sem_z)

        cnt_a = jnp.minimum(nbw, hb)
        cnt_b = nbw - cnt_a

        def fetch_idx(first_b, cnt):
            def f(k, carry):
                pltpu.async_copy(
                    edge.at[:, pl.ds((first_b + k * ICH) * B, ICH * B)],
                    eb.at[:, pl.ds(k * ICH * B, ICH * B)], sem_d)
                return carry
            lax.fori_loop(0, cnt // ICH, f, 0)

            def fd(k, carry):
                pltpu.make_async_copy(
                    edge.at[:, pl.ds(0, ICH * B)],
                    eb.at[:, pl.ds(0, ICH * B)], sem_d).wait()
                return carry
            lax.fori_loop(0, cnt // ICH, fd, 0)

        def prime_ring(cnt, lo, hi):
            for k in range(lo, hi):
                @pl.when(k < cnt)
                def _():
                    pltpu.async_copy(table.at[eb.at[0, pl.ds(k * B, B)]],
                                     rows.at[k], sem_g)

        def run_batches(cnt):
            def body(i, carry):
                b0 = 4 * i
                for k in range(4):
                    pltpu.make_async_copy(table.at[eb.at[0, pl.ds(0, B)]],
                                          rows.at[k], sem_g).wait()
                    pltpu.async_copy(
                        rows.at[k],
                        acc_sh.at[eb.at[1, pl.ds((b0 + k) * B, B)]],
                        sem_s, add=True)
                    pltpu.async_copy(
                        ones_v,
                        deg_sh.at[eb.at[1, pl.ds((b0 + k) * B, B)]],
                        sem_d, add=True)
                for k in range(4):
                    pltpu.make_async_copy(rows.at[k], acc_sh.at[pl.ds(0, B)],
                                          sem_s).wait()
                    pltpu.make_async_copy(ones_v, deg_sh.at[pl.ds(0, B)],
                                          sem_d).wait()

                    @pl.when(b0 + 4 + k < cnt)
                    def _():
                        pltpu.async_copy(
                            table.at[eb.at[0, pl.ds((b0 + 4 + k) * B, B)]],
                            rows.at[k], sem_g)
                return carry

            lax.fori_loop(0, cnt // 4, body, 0)

        fetch_idx(start_b, cnt_a)
        prime_ring(cnt_a, 0, 3)
        for k in range(rpt // B):
            pltpu.make_async_copy(rows.at[3], acc_sh.at[pl.ds(base_r, B)],
                                  sem_z).wait()
        if rpt % B:
            pltpu.make_async_copy(
                rows.at[3].at[pl.ds(0, rpt % B)],
                acc_sh.at[pl.ds(base_r, rpt % B)], sem_z).wait()

        @pl.when(s < dtiles)
        def _():
            pltpu.make_async_copy(zdeg, deg_sh.at[pl.ds(0, dchunk)],
                                  sem_z).wait()
        prime_ring(cnt_a, 3, 4)
        plsc.subcore_barrier()
        run_batches(cnt_a)
        if hb < nb_max:
            fetch_idx(start_b + hb, cnt_b)
            prime_ring(cnt_b, 0, 4)
            run_batches(cnt_b)
        plsc.subcore_barrier()

        pltpu.sync_copy(acc_sh.at[pl.ds(base_r, rpt)],
                        acc_out.at[c].at[pl.ds(base_r, rpt)])

        @pl.when(s < dtiles)
        def _():
            pltpu.sync_copy(deg_sh.at[pl.ds(s * dchunk, dchunk)],
                            deg_out.at[c].at[pl.ds(s * dchunk, dchunk)])

    return seg


def _make_pair_gather(n_table, ep):
    assert ep == (ep // B) * B and ep // B == NW

    @functools.partial(
        pl.kernel,
        out_type=jax.ShapeDtypeStruct((2 * ep, D), jnp.float32),
        mesh=_mesh(),
        scratch_types=(
            pltpu.VMEM((2, B), jnp.int32),
            pltpu.VMEM((B, D), jnp.float32),
            pltpu.VMEM((B, D), jnp.float32),
            pltpu.SemaphoreType.DMA,
            pltpu.SemaphoreType.DMA,
        ),
    )
    def gat(table, pair, out, eb, row_u, row_v, sem_u, sem_v):
        c = lax.axis_index("c")
        s = lax.axis_index("s")
        w = s * NCSC + c
        pltpu.sync_copy(pair.at[:, pl.ds(w * B, B)], eb)
        pltpu.async_copy(table.at[eb.at[0]], row_u, sem_u)
        pltpu.async_copy(table.at[eb.at[1]], row_v, sem_v)
        pltpu.make_async_copy(table.at[eb.at[0]], row_u, sem_u).wait()
        pltpu.sync_copy(row_u, out.at[pl.ds(w * B, B)])
        pltpu.make_async_copy(table.at[eb.at[1]], row_v, sem_v).wait()
        pltpu.sync_copy(row_v, out.at[pl.ds(ep + w * B, B)])

    return gat


def _self_proj(h_prev, w_self, b, n_dst):
    def body(h_ref, ws_ref, b_ref, out_ref):
        out_ref[...] = (jnp.dot(h_ref[:n_dst, :], ws_ref[...],
                                preferred_element_type=jnp.float32)
                        + b_ref[...])

    return pl.pallas_call(
        body,
        out_shape=jax.ShapeDtypeStruct((n_dst, D), jnp.float32),
    )(h_prev, w_self, b)


def _sage_post(acc, deg, zs, w_neigh, g, beta, n_dst):
    def body(acc_ref, deg_ref, zs_ref, wn_ref, g_ref, beta_ref, out_ref):
        agg = acc_ref[0, :n_dst, :] + acc_ref[1, :n_dst, :]
        dg = deg_ref[0, :n_dst] + deg_ref[1, :n_dst]
        mean = agg / jnp.maximum(dg, 1.0)[:, None]
        z = zs_ref[...] + jnp.dot(mean, wn_ref[...],
                                  preferred_element_type=jnp.float32)
        mu = jnp.mean(z, axis=0)
        var = jnp.mean((z - mu) ** 2, axis=0)
        zn = (z - mu) * jax.lax.rsqrt(var + 1e-5) * g_ref[...] + beta_ref[...]
        out_ref[...] = jnp.maximum(zn, 0.0)

    return pl.pallas_call(
        body,
        out_shape=jax.ShapeDtypeStruct((n_dst, D), jnp.float32),
    )(acc, deg, zs, w_neigh, g, beta)


def _edge_mlp(huv, e_feat_t, wm1, bm1, wm2, bm2, n_pairs, n_cls):
    def body(huv_ref, eft_ref, w1_ref, b1_ref, w2_ref, b2_ref, out_ref):
        hu = huv_ref[:n_pairs, :]
        hv = huv_ref[n_pairs:, :]
        t = (jnp.dot(hu, w1_ref[:D, :], preferred_element_type=jnp.float32)
             + jnp.dot(hv, w1_ref[D:2 * D, :],
                       preferred_element_type=jnp.float32)
             + lax.dot_general(eft_ref[...], w1_ref[2 * D:, :],
                               (((0,), (0,)), ((), ())),
                               preferred_element_type=jnp.float32)
             + b1_ref[...])
        t = jnp.maximum(t, 0.0)
        out_ref[...] = (lax.dot_general(w2_ref[...], t,
                                        (((0,), (1,)), ((), ())),
                                        preferred_element_type=jnp.float32)
                        + b2_ref[...][:, None])

    return pl.pallas_call(
        body,
        out_shape=jax.ShapeDtypeStruct((n_cls, n_pairs), jnp.float32),
    )(huv, e_feat_t, wm1, bm1, wm2, bm2)


def kernel(x_nodes, e_feat, W_self0, W_neigh0, b0, g0, beta0,
           W_self1, W_neigh1, b1, g1, beta1, Wm1, bm1, Wm2, bm2,
           edge_index0, edge_index1, pair_edges):
    n0 = x_nodes.shape[0]
    e0 = edge_index0.shape[1]
    e1 = edge_index1.shape[1]
    nd0 = 5000
    nd0p = 5120
    nd1 = 2048
    ep = pair_edges.shape[1]

    zs0 = _self_proj(x_nodes, W_self0, b0, nd0)
    acc0, deg0 = _make_segsum(n0, nd0p, e0 // B, 80)(x_nodes, edge_index0)
    h0 = _sage_post(acc0, deg0, zs0, W_neigh0, g0, beta0, nd0)

    zs1 = _self_proj(h0, W_self1, b1, nd1)
    acc1, deg1 = _make_segsum(nd0, nd1, e1 // B, 16)(h0, edge_index1)
    h1 = _sage_post(acc1, deg1, zs1, W_neigh1, g1, beta1, nd1)

    huv = _make_pair_gather(nd1, ep)(h1, pair_edges)
    out_t = _edge_mlp(huv, e_feat.T, Wm1, bm1, Wm2, bm2, ep, Wm2.shape[1])
    return out_t.T
